# baseline (speedup 1.0000x reference)
import numpy as np
import jax
import jax.numpy as jnp
from jax import lax
from jax.experimental import pallas as pl
from jax.experimental.pallas import tpu as pltpu
from jax.experimental.pallas import tpu_sc as plsc

_B, _C, _H, _W, _Q = 32, 1, 64, 256, 1024
_D = _C * _H * _W
_NC, _NS = 2, 16
_NW = _NC * _NS
_RPW = _Q // _NW
_CK = 4
_NCH = _RPW // _CK

_PERM = np.random.default_rng(1).permutation(_Q).astype(np.int32)


def _build_gidx() -> np.ndarray:
    g = np.zeros((_NW, _NCH + 1, 16), np.int32)
    for w in range(_NW):
        for c in range(_NCH):
            g[w, c, :_CK] = _PERM[w * _RPW + c * _CK : w * _RPW + (c + 1) * _CK]
        g[w, _NCH, 0] = _PERM[w]
    return g.reshape(-1)


_GIDX_NP = _build_gidx()


def _sc_body(qq, qk, qry, keyb, gidx, dq_q, dq_k, out_q, out_k,
             idx_v, buf, sem):
    wid = lax.axis_index("s") * _NC + lax.axis_index("c")
    base = wid * _RPW

    def do_queue(src_tbl, batch, out, dq):
        pltpu.sync_copy(gidx.at[pl.ds((wid * (_NCH + 1) + _NCH) * 16, 16)], idx_v)
        pltpu.async_copy(src_tbl.at[idx_v.at[pl.ds(0, 1)]],
                         buf.at[pl.ds(0, 1)], sem).wait()
        pltpu.sync_copy(buf.at[pl.ds(0, 1)], dq.at[pl.ds(wid, 1)])

        @pl.when(wid == 0)
        def _():
            for c in range(_NCH):
                pltpu.sync_copy(batch.at[pl.ds(c * _CK, _CK)],
                                buf.at[pl.ds(0, _CK)])
                pltpu.sync_copy(buf.at[pl.ds(0, _CK)],
                                out.at[pl.ds(c * _CK, _CK)])

        @pl.when(wid != 0)
        def _():
            for c in range(_NCH):
                pltpu.sync_copy(gidx.at[pl.ds((wid * (_NCH + 1) + c) * 16, 16)],
                                idx_v)
                pltpu.async_copy(src_tbl.at[idx_v.at[pl.ds(0, _CK)]],
                                 buf.at[pl.ds(0, _CK)], sem).wait()
                pltpu.sync_copy(buf.at[pl.ds(0, _CK)],
                                out.at[pl.ds(base + c * _CK, _CK)])

    do_queue(qq, qry, out_q, dq_q)
    do_queue(qk, keyb, out_k, dq_k)


def kernel(queue_q, queue_k, query, key):
    qq = queue_q.reshape(_Q, _D)
    qk = queue_k.reshape(_Q, _D)
    qry = query.reshape(_B, _D)
    kb = key.reshape(_B, _D)
    gidx = jnp.asarray(_GIDX_NP)

    mesh = plsc.VectorSubcoreMesh(core_axis_name="c", subcore_axis_name="s")
    f32 = jnp.float32
    sc_call = pl.kernel(
        _sc_body,
        mesh=mesh,
        compiler_params=pltpu.CompilerParams(use_tc_tiling_on_sc=False),
        out_type=[
            jax.ShapeDtypeStruct((_B, _D), f32),
            jax.ShapeDtypeStruct((_B, _D), f32),
            jax.ShapeDtypeStruct((_Q, _D), f32),
            jax.ShapeDtypeStruct((_Q, _D), f32),
        ],
        scratch_types=[
            pltpu.VMEM((16,), jnp.int32),
            pltpu.VMEM((_CK, _D), f32),
            pltpu.SemaphoreType.DMA,
        ],
    )
    dq_q, dq_k, nq, nk = sc_call(qq, qk, qry, kb, gidx)
    return (
        dq_q.reshape(_B, _C, _H, _W),
        dq_k.reshape(_B, _C, _H, _W),
        nq.reshape(_Q, _C, _H, _W),
        nk.reshape(_Q, _C, _H, _W),
    )


# speedup vs baseline: 1.0326x; 1.0326x over previous
import numpy as np
import jax
import jax.numpy as jnp
from jax import lax
from jax.experimental import pallas as pl
from jax.experimental.pallas import tpu as pltpu
from jax.experimental.pallas import tpu_sc as plsc

_B, _C, _H, _W, _Q = 32, 1, 64, 256, 1024
_D = _C * _H * _W
_NC, _NS = 2, 16
_NW = _NC * _NS
_RPW = _Q // _NW

_CHUNKS = [4, 3, 4, 3, 4, 3, 4, 3, 4]
_NSLOT = len(_CHUNKS) + 1

_PERM = np.random.default_rng(1).permutation(_Q).astype(np.int32)


def _build_gidx() -> np.ndarray:
    g = np.zeros((_NW, _NSLOT, 16), np.int32)
    for w in range(_NW):
        off = 0
        for c, ck in enumerate(_CHUNKS):
            g[w, c, :ck] = _PERM[w * _RPW + off : w * _RPW + off + ck]
            off += ck
        g[w, _NSLOT - 1, 0] = _PERM[w]
    return g.reshape(-1)


_GIDX_NP = _build_gidx()


def _run_jobs(jobs):
    n = len(jobs)
    gh = [None] * n
    sh = [None] * n
    gh[0] = jobs[0][0]()
    for t in range(n):
        if t + 1 < n:
            if t - 1 >= 0:
                sh[t - 1].wait()
            gh[t + 1] = jobs[t + 1][0]()
        gh[t].wait()
        sh[t] = jobs[t][1]()
    if n >= 2:
        sh[n - 2].wait()
    sh[n - 1].wait()


def _sc_body(qq, qk, qry, keyb, gidx, dq_q, dq_k, out_q, out_k,
             idxv, buf_a, buf_b, ga, gb, sa, sb):
    wid = lax.axis_index("s") * _NC + lax.axis_index("c")
    base = wid * _RPW
    bufs = (buf_a, buf_b)
    gsems = (ga, gb)
    ssems = (sa, sb)

    pltpu.sync_copy(gidx.at[pl.ds(wid * _NSLOT * 16, _NSLOT * 16)], idxv)

    def gather_job(t, tbl, slot, ck):
        buf, sem = bufs[t % 2], gsems[t % 2]
        return lambda: pltpu.async_copy(
            tbl.at[idxv.at[pl.ds(slot * 16, ck)]], buf.at[pl.ds(0, ck)], sem)

    def linear_in_job(t, src, off, ck):
        buf, sem = bufs[t % 2], gsems[t % 2]
        return lambda: pltpu.async_copy(
            src.at[pl.ds(off, ck)], buf.at[pl.ds(0, ck)], sem)

    def scatter_job(t, dst, off, ck):
        buf, sem = bufs[t % 2], ssems[t % 2]
        return lambda: pltpu.async_copy(
            buf.at[pl.ds(0, ck)], dst.at[pl.ds(off, ck)], sem)

    def queue_jobs(t0, src_tbl, batch, out, dq, linear_batch):
        jobs = []
        t = t0
        off = 0
        for c, ck in enumerate(_CHUNKS):
            if linear_batch:
                gj = linear_in_job(t, batch, off, ck)
            else:
                gj = gather_job(t, src_tbl, c, ck)
            jobs.append((gj, scatter_job(t, out, base + off, ck)))
            t += 1
            off += ck
        jobs.append((gather_job(t, src_tbl, _NSLOT - 1, 1),
                     scatter_job(t, dq, wid, 1)))
        return jobs

    @pl.when(wid == 0)
    def _():
        _run_jobs(queue_jobs(0, qq, qry, out_q, dq_q, True)
                  + queue_jobs(_NSLOT, qk, keyb, out_k, dq_k, True))

    @pl.when(wid != 0)
    def _():
        _run_jobs(queue_jobs(0, qq, qry, out_q, dq_q, False)
                  + queue_jobs(_NSLOT, qk, keyb, out_k, dq_k, False))


def kernel(queue_q, queue_k, query, key):
    qq = queue_q.reshape(_Q, _D)
    qk = queue_k.reshape(_Q, _D)
    qry = query.reshape(_B, _D)
    kb = key.reshape(_B, _D)
    gidx = jnp.asarray(_GIDX_NP)

    mesh = plsc.VectorSubcoreMesh(core_axis_name="c", subcore_axis_name="s")
    f32 = jnp.float32
    sc_call = pl.kernel(
        _sc_body,
        mesh=mesh,
        compiler_params=pltpu.CompilerParams(use_tc_tiling_on_sc=False),
        out_type=[
            jax.ShapeDtypeStruct((_B, _D), f32),
            jax.ShapeDtypeStruct((_B, _D), f32),
            jax.ShapeDtypeStruct((_Q, _D), f32),
            jax.ShapeDtypeStruct((_Q, _D), f32),
        ],
        scratch_types=[
            pltpu.VMEM((_NSLOT * 16,), jnp.int32),
            pltpu.VMEM((4, _D), f32),
            pltpu.VMEM((3, _D), f32),
            pltpu.SemaphoreType.DMA,
            pltpu.SemaphoreType.DMA,
            pltpu.SemaphoreType.DMA,
            pltpu.SemaphoreType.DMA,
        ],
    )
    dq_q, dq_k, nq, nk = sc_call(qq, qk, qry, kb, gidx)
    return (
        dq_q.reshape(_B, _C, _H, _W),
        dq_k.reshape(_B, _C, _H, _W),
        nq.reshape(_Q, _C, _H, _W),
        nk.reshape(_Q, _C, _H, _W),
    )


# speedup vs baseline: 3.2631x; 3.1599x over previous
import numpy as np
import jax
import jax.numpy as jnp
from jax import lax
from jax.experimental import pallas as pl
from jax.experimental.pallas import tpu as pltpu
from jax.experimental.pallas import tpu_sc as plsc

_B, _C, _H, _W, _Q = 32, 1, 64, 256, 1024
_D = _C * _H * _W
_NC, _NS = 2, 16
_NW = _NC * _NS
_RPW = _Q // _NW

_CHUNKS = [4, 3, 4, 3, 4, 3, 4, 3, 4]
_NSLOT = len(_CHUNKS) + 1

_PERM = np.random.default_rng(1).permutation(_Q).astype(np.int32)


def _build_gidx() -> np.ndarray:
    g = np.zeros((_NW, _NSLOT, 16), np.int32)
    for w in range(_NW):
        off = 0
        for c, ck in enumerate(_CHUNKS):
            g[w, c, :ck] = _PERM[w * _RPW + off : w * _RPW + off + ck]
            off += ck
        g[w, _NSLOT - 1, 0] = _PERM[w]
    return g.reshape(-1)


_GIDX_NP = _build_gidx()


def _run_jobs(jobs):
    n = len(jobs)
    gh = [None] * n
    sh = [None] * n
    gh[0] = jobs[0][0]()
    for t in range(n):
        if t + 1 < n:
            if t - 1 >= 0:
                sh[t - 1].wait()
            gh[t + 1] = jobs[t + 1][0]()
        gh[t].wait()
        sh[t] = jobs[t][1]()
    if n >= 2:
        sh[n - 2].wait()
    sh[n - 1].wait()


def _sc_body(qq, qk, qry, keyb, gidx, dq_q, dq_k, out_q, out_k,
             idxv, buf_a, buf_b, ga, gb, sa, sb):
    wid = lax.axis_index("s") * _NC + lax.axis_index("c")
    base = wid * _RPW
    bufs = (buf_a, buf_b)
    gsems = (ga, gb)
    ssems = (sa, sb)

    pltpu.sync_copy(gidx.at[pl.ds(wid * _NSLOT * 16, _NSLOT * 16)], idxv)

    def gather_job(t, tbl, slot, ck):
        buf, sem = bufs[t % 2], gsems[t % 2]
        return lambda: pltpu.async_copy(
            tbl.at[idxv.at[pl.ds(slot * 16, ck)]], buf.at[pl.ds(0, ck)], sem)

    def linear_in_job(t, src, off, ck):
        buf, sem = bufs[t % 2], gsems[t % 2]
        return lambda: pltpu.async_copy(
            src.at[pl.ds(off, ck)], buf.at[pl.ds(0, ck)], sem)

    def scatter_job(t, dst, off, ck):
        buf, sem = bufs[t % 2], ssems[t % 2]
        return lambda: pltpu.async_copy(
            buf.at[pl.ds(0, ck)], dst.at[pl.ds(off, ck)], sem)

    def queue_jobs(t0, src_tbl, batch, out, dq, linear_batch):
        jobs = []
        t = t0
        off = 0
        for c, ck in enumerate(_CHUNKS):
            if linear_batch:
                gj = linear_in_job(t, batch, off, ck)
            else:
                gj = gather_job(t, src_tbl, c, ck)
            jobs.append((gj, scatter_job(t, out, base + off, ck)))
            t += 1
            off += ck
        jobs.append((gather_job(t, src_tbl, _NSLOT - 1, 1),
                     scatter_job(t, dq, wid, 1)))
        return jobs

    @pl.when(wid == 0)
    def _():
        _run_jobs(queue_jobs(0, qq, qry, out_q, dq_q, True)
                  + queue_jobs(_NSLOT, qk, keyb, out_k, dq_k, True))

    @pl.when(wid != 0)
    def _():
        _run_jobs(queue_jobs(0, qq, qry, out_q, dq_q, False)
                  + queue_jobs(_NSLOT, qk, keyb, out_k, dq_k, False))


def kernel(queue_q, queue_k, query, key):
    qq = queue_q
    qk = queue_k
    qry = query
    kb = key
    gidx = jnp.asarray(_GIDX_NP)

    mesh = plsc.VectorSubcoreMesh(core_axis_name="c", subcore_axis_name="s")
    f32 = jnp.float32
    sc_call = pl.kernel(
        _sc_body,
        mesh=mesh,
        out_type=[
            jax.ShapeDtypeStruct((_B, _C, _H, _W), f32),
            jax.ShapeDtypeStruct((_B, _C, _H, _W), f32),
            jax.ShapeDtypeStruct((_Q, _C, _H, _W), f32),
            jax.ShapeDtypeStruct((_Q, _C, _H, _W), f32),
        ],
        scratch_types=[
            pltpu.VMEM((_NSLOT * 16,), jnp.int32),
            pltpu.VMEM((4, _C, _H, _W), f32),
            pltpu.VMEM((3, _C, _H, _W), f32),
            pltpu.SemaphoreType.DMA,
            pltpu.SemaphoreType.DMA,
            pltpu.SemaphoreType.DMA,
            pltpu.SemaphoreType.DMA,
        ],
    )
    dq_q, dq_k, nq, nk = sc_call(qq, qk, qry, kb, gidx)
    return (dq_q, dq_k, nq, nk)
